# baseline (speedup 1.0000x reference)
import jax
import jax.numpy as jnp
from jax.experimental import pallas as pl
from jax.experimental.pallas import tpu as pltpu

PSZ = 14
NP = 16
NPP = 17
CH = 96
NEXP = 8
CB = 16
NCB = CH // CB
NPAT = NPP * NPP
TL = PSZ * PSZ


def _patchify_kernel(x_ref, t_ref, m_ref):
    xb = x_ref[0]
    kh_i = jax.lax.broadcasted_iota(jnp.int32, (PSZ, TL), 0)
    q_i = jax.lax.broadcasted_iota(jnp.int32, (PSZ, TL), 1)
    rep = (q_i % PSZ == kh_i).astype(jnp.float32)
    kv_i = jax.lax.broadcasted_iota(jnp.int32, (PSZ, TL), 0)
    sel = (q_i // PSZ == kv_i).astype(jnp.float32)[None]
    zt = jnp.zeros((CB, 1, TL), jnp.float32)
    for pv in range(NP):
        rp = xb[:, pv * PSZ:(pv + 1) * PSZ, :]
        rows = []
        for ph in range(NP):
            vk = rp[:, :, ph * PSZ:(ph + 1) * PSZ].reshape(CB * PSZ, PSZ)
            tmp = jax.lax.dot_general(
                vk, rep, (((1,), (0,)), ((), ())),
                preferred_element_type=jnp.float32)
            tmp = tmp.reshape(CB, PSZ, TL) * sel
            rows.append(tmp.sum(axis=1, keepdims=True))
        tiles = jnp.concatenate(rows, axis=1)
        t_ref[0, :, pv * NPP:pv * NPP + NP, :] = tiles
        t_ref[0, :, pv * NPP + NP:(pv + 1) * NPP, :] = zt
    t_ref[0, :, NP * NPP:NPAT, :] = jnp.zeros((CB, NPAT - NP * NPP, TL),
                                              jnp.float32)
    xr = xb.reshape(CB, NP, PSZ, 224)
    s_row = xr.sum(axis=2)
    z1 = jnp.zeros((CB, 1, 224), jnp.float32)
    z2 = jnp.zeros((CB, 2, 224), jnp.float32)
    w = (jnp.concatenate([s_row, z1], axis=1)
         + jnp.concatenate([z1, xr[:, :, PSZ - 1, :]], axis=1)
         + jnp.concatenate([xr[:, 1:, 0, :], z2], axis=1))
    xc = w.reshape(CB, NPP, NP, PSZ)
    s_col = xc.sum(axis=3)
    c1 = jnp.zeros((CB, NPP, 1), jnp.float32)
    c2 = jnp.zeros((CB, NPP, 2), jnp.float32)
    m = (jnp.concatenate([s_col, c1], axis=2)
         + jnp.concatenate([c1, xc[:, :, :, PSZ - 1]], axis=2)
         + jnp.concatenate([xc[:, :, 1:, 0], c2], axis=2))
    m_ref[0] = m * (1.0 / 256.0)


def _gate_kernel(m_ref, wg_ref, bg_ref, s_ref, e_ref):
    pooled = m_ref[0]
    logits = jax.lax.dot_general(
        pooled, wg_ref[...], (((1,), (0,)), ((), ())),
        preferred_element_type=jnp.float32)
    logits = logits + bg_ref[...]
    mx = jnp.max(logits, axis=1, keepdims=True)
    s_ref[0, 0, :] = 1.0 / jnp.sum(jnp.exp(logits - mx), axis=1)
    e_ref[0, 0, :] = jnp.argmax(logits, axis=1).astype(jnp.int32)


GRP = 17


def _apply_kernel(s_sm, e_sm, t_ref, we_ref, be_ref, y_ref):
    base = pl.program_id(0) * NPAT + pl.program_id(1) * GRP
    for g in range(GRP):
        idx = e_sm[base + g]
        sv = s_sm[base + g]
        y = jax.lax.dot_general(
            we_ref[idx], t_ref[0, g], (((1,), (0,)), ((), ())),
            preferred_element_type=jnp.float32)
        y_ref[0, g] = (y + be_ref[idx][:, None]) * sv


def _rearrange_kernel(y_ref, out_ref):
    v = y_ref[0, :, 0, :NP, :]
    kh_i = jax.lax.broadcasted_iota(jnp.int32, (PSZ, NP * PSZ), 0)
    j_i = jax.lax.broadcasted_iota(jnp.int32, (PSZ, NP * PSZ), 1)
    rep = (j_i % PSZ == kh_i).astype(jnp.float32)
    nh_i = jax.lax.broadcasted_iota(jnp.int32, (NP, NP * PSZ), 0)
    sel = (j_i[0:1] // PSZ == nh_i).astype(jnp.float32)[None]
    for kv in range(PSZ):
        vk = v[:, :, kv * PSZ:(kv + 1) * PSZ].reshape(CH * NP, PSZ)
        tmp = jax.lax.dot_general(
            vk, rep, (((1,), (0,)), ((), ())),
            preferred_element_type=jnp.float32)
        tmp = tmp.reshape(CH, NP, NP * PSZ) * sel
        out_ref[0, :, 0, kv, :] = tmp.sum(axis=1)


def kernel(x, Wg, bg, We, be):
    B = x.shape[0]
    t, m = pl.pallas_call(
        _patchify_kernel,
        grid=(B, NCB),
        in_specs=[pl.BlockSpec((1, CB, 224, 224),
                               lambda b, cb: (b, cb, 0, 0))],
        out_specs=[
            pl.BlockSpec((1, CB, NPAT, TL), lambda b, cb: (b, cb, 0, 0)),
            pl.BlockSpec((1, CB, NPP, NPP), lambda b, cb: (b, cb, 0, 0)),
        ],
        out_shape=[
            jax.ShapeDtypeStruct((B, CH, NPAT, TL), jnp.float32),
            jax.ShapeDtypeStruct((B, CH, NPP, NPP), jnp.float32),
        ],
    )(x)

    s3, e3 = pl.pallas_call(
        _gate_kernel,
        grid=(B,),
        in_specs=[
            pl.BlockSpec((1, NPAT, CH), lambda b: (b, 0, 0)),
            pl.BlockSpec((CH, NEXP), lambda b: (0, 0)),
            pl.BlockSpec((1, NEXP), lambda b: (0, 0)),
        ],
        out_specs=[
            pl.BlockSpec((1, 1, NPAT), lambda b: (b, 0, 0)),
            pl.BlockSpec((1, 1, NPAT), lambda b: (b, 0, 0)),
        ],
        out_shape=[
            jax.ShapeDtypeStruct((B, 1, NPAT), jnp.float32),
            jax.ShapeDtypeStruct((B, 1, NPAT), jnp.int32),
        ],
    )(m.reshape(B, NPAT, CH), Wg, bg.reshape(1, NEXP))

    y = pl.pallas_call(
        _apply_kernel,
        grid_spec=pltpu.PrefetchScalarGridSpec(
            num_scalar_prefetch=2,
            grid=(B, NPAT // GRP),
            in_specs=[
                pl.BlockSpec((1, GRP, CH, TL), lambda b, i, *_: (b, i, 0, 0)),
                pl.BlockSpec((NEXP, CH, CH), lambda b, i, *_: (0, 0, 0)),
                pl.BlockSpec((NEXP, CH), lambda b, i, *_: (0, 0)),
            ],
            out_specs=pl.BlockSpec((1, GRP, CH, TL), lambda b, i, *_: (b, i, 0, 0)),
        ),
        out_shape=jax.ShapeDtypeStruct((B, NPAT, CH, TL), jnp.float32),
    )(s3.reshape(B * NPAT), e3.reshape(B * NPAT),
      t.reshape(B, NPAT, CH, TL), We, be)

    out = pl.pallas_call(
        _rearrange_kernel,
        grid=(B, NP),
        in_specs=[
            pl.BlockSpec((1, CH, 1, NPP, TL), lambda b, nv: (b, 0, nv, 0, 0)),
        ],
        out_specs=pl.BlockSpec((1, CH, 1, PSZ, NP * PSZ),
                               lambda b, nv: (b, 0, nv, 0, 0)),
        out_shape=jax.ShapeDtypeStruct((B, CH, NP, PSZ, NP * PSZ), jnp.float32),
    )(y.reshape(B, CH, NPP, NPP, TL))
    return out.reshape(B, CH, NP * PSZ, NP * PSZ)


# speedup vs baseline: 1.1105x; 1.1105x over previous
import jax
import jax.numpy as jnp
from jax.experimental import pallas as pl
from jax.experimental.pallas import tpu as pltpu

PSZ = 14
NP = 16
NPP = 17
CH = 96
NEXP = 8
CB = 16
NCB = CH // CB
NPAT = NPP * NPP
TL = PSZ * PSZ


def _patchify_kernel(x_ref, t_ref, m_ref):
    xb = x_ref[0]
    zt = jnp.zeros((CB, 1, TL), jnp.float32)
    for pv in range(NP):
        rp = xb[:, pv * PSZ:(pv + 1) * PSZ, :]
        tiles = rp.reshape(CB, PSZ, NP, PSZ).transpose(0, 2, 1, 3)
        tiles = tiles.reshape(CB, NP, TL)
        t_ref[0, :, pv * NPP:pv * NPP + NP, :] = tiles
        t_ref[0, :, pv * NPP + NP:(pv + 1) * NPP, :] = zt
    t_ref[0, :, NP * NPP:NPAT, :] = jnp.zeros((CB, NPAT - NP * NPP, TL),
                                              jnp.float32)
    xr = xb.reshape(CB, NP, PSZ, 224)
    s_row = xr.sum(axis=2)
    z1 = jnp.zeros((CB, 1, 224), jnp.float32)
    z2 = jnp.zeros((CB, 2, 224), jnp.float32)
    w = (jnp.concatenate([s_row, z1], axis=1)
         + jnp.concatenate([z1, xr[:, :, PSZ - 1, :]], axis=1)
         + jnp.concatenate([xr[:, 1:, 0, :], z2], axis=1))
    xc = w.reshape(CB, NPP, NP, PSZ)
    s_col = xc.sum(axis=3)
    c1 = jnp.zeros((CB, NPP, 1), jnp.float32)
    c2 = jnp.zeros((CB, NPP, 2), jnp.float32)
    m = (jnp.concatenate([s_col, c1], axis=2)
         + jnp.concatenate([c1, xc[:, :, :, PSZ - 1]], axis=2)
         + jnp.concatenate([xc[:, :, 1:, 0], c2], axis=2))
    m_ref[0] = m * (1.0 / 256.0)


def _gate_kernel(m_ref, wg_ref, bg_ref, s_ref, e_ref):
    pooled = m_ref[0]
    logits = jax.lax.dot_general(
        pooled, wg_ref[...], (((1,), (0,)), ((), ())),
        preferred_element_type=jnp.float32)
    logits = logits + bg_ref[...]
    mx = jnp.max(logits, axis=1, keepdims=True)
    s_ref[0, 0, :] = 1.0 / jnp.sum(jnp.exp(logits - mx), axis=1)
    e_ref[0, 0, :] = jnp.argmax(logits, axis=1).astype(jnp.int32)


GRP = 17


def _apply_kernel(s_sm, e_sm, t_ref, we_ref, be_ref, y_ref):
    base = pl.program_id(0) * NPAT + pl.program_id(1) * GRP
    for g in range(GRP):
        idx = e_sm[base + g]
        sv = s_sm[base + g]
        y = jax.lax.dot_general(
            we_ref[idx], t_ref[0, g], (((1,), (0,)), ((), ())),
            preferred_element_type=jnp.float32)
        y_ref[0, g] = (y + be_ref[idx][:, None]) * sv


def _rearrange_kernel(y_ref, out_ref):
    v = y_ref[0, :, 0, :NP, :]
    kh_i = jax.lax.broadcasted_iota(jnp.int32, (PSZ, NP * PSZ), 0)
    j_i = jax.lax.broadcasted_iota(jnp.int32, (PSZ, NP * PSZ), 1)
    rep = (j_i % PSZ == kh_i).astype(jnp.float32)
    nh_i = jax.lax.broadcasted_iota(jnp.int32, (NP, NP * PSZ), 0)
    sel = (j_i[0:1] // PSZ == nh_i).astype(jnp.float32)[None]
    for kv in range(PSZ):
        vk = v[:, :, kv * PSZ:(kv + 1) * PSZ].reshape(CH * NP, PSZ)
        tmp = jax.lax.dot_general(
            vk, rep, (((1,), (0,)), ((), ())),
            preferred_element_type=jnp.float32)
        tmp = tmp.reshape(CH, NP, NP * PSZ) * sel
        out_ref[0, :, 0, kv, :] = tmp.sum(axis=1)


def kernel(x, Wg, bg, We, be):
    B = x.shape[0]
    t, m = pl.pallas_call(
        _patchify_kernel,
        grid=(B, NCB),
        in_specs=[pl.BlockSpec((1, CB, 224, 224),
                               lambda b, cb: (b, cb, 0, 0))],
        out_specs=[
            pl.BlockSpec((1, CB, NPAT, TL), lambda b, cb: (b, cb, 0, 0)),
            pl.BlockSpec((1, CB, NPP, NPP), lambda b, cb: (b, cb, 0, 0)),
        ],
        out_shape=[
            jax.ShapeDtypeStruct((B, CH, NPAT, TL), jnp.float32),
            jax.ShapeDtypeStruct((B, CH, NPP, NPP), jnp.float32),
        ],
    )(x)

    s3, e3 = pl.pallas_call(
        _gate_kernel,
        grid=(B,),
        in_specs=[
            pl.BlockSpec((1, NPAT, CH), lambda b: (b, 0, 0)),
            pl.BlockSpec((CH, NEXP), lambda b: (0, 0)),
            pl.BlockSpec((1, NEXP), lambda b: (0, 0)),
        ],
        out_specs=[
            pl.BlockSpec((1, 1, NPAT), lambda b: (b, 0, 0)),
            pl.BlockSpec((1, 1, NPAT), lambda b: (b, 0, 0)),
        ],
        out_shape=[
            jax.ShapeDtypeStruct((B, 1, NPAT), jnp.float32),
            jax.ShapeDtypeStruct((B, 1, NPAT), jnp.int32),
        ],
    )(m.reshape(B, NPAT, CH), Wg, bg.reshape(1, NEXP))

    y = pl.pallas_call(
        _apply_kernel,
        grid_spec=pltpu.PrefetchScalarGridSpec(
            num_scalar_prefetch=2,
            grid=(B, NPAT // GRP),
            in_specs=[
                pl.BlockSpec((1, GRP, CH, TL), lambda b, i, *_: (b, i, 0, 0)),
                pl.BlockSpec((NEXP, CH, CH), lambda b, i, *_: (0, 0, 0)),
                pl.BlockSpec((NEXP, CH), lambda b, i, *_: (0, 0)),
            ],
            out_specs=pl.BlockSpec((1, GRP, CH, TL), lambda b, i, *_: (b, i, 0, 0)),
        ),
        out_shape=jax.ShapeDtypeStruct((B, NPAT, CH, TL), jnp.float32),
    )(s3.reshape(B * NPAT), e3.reshape(B * NPAT),
      t.reshape(B, NPAT, CH, TL), We, be)

    out = pl.pallas_call(
        _rearrange_kernel,
        grid=(B, NP),
        in_specs=[
            pl.BlockSpec((1, CH, 1, NPP, TL), lambda b, nv: (b, 0, nv, 0, 0)),
        ],
        out_specs=pl.BlockSpec((1, CH, 1, PSZ, NP * PSZ),
                               lambda b, nv: (b, 0, nv, 0, 0)),
        out_shape=jax.ShapeDtypeStruct((B, CH, NP, PSZ, NP * PSZ), jnp.float32),
    )(y.reshape(B, CH, NPP, NPP, TL))
    return out.reshape(B, CH, NP * PSZ, NP * PSZ)
